# Initial kernel scaffold; baseline (speedup 1.0000x reference)
import jax
import jax.numpy as jnp
from jax import lax
from jax.experimental import pallas as pl
from jax.experimental.pallas import tpu as pltpu
from jax.experimental.pallas import tpu_sc as plsc

N_NODES = 10000
N_EDGES = 320000
D_FEAT = 128

NUM_CORES = 2
NUM_SUBCORES = 16
NUM_TILES = NUM_CORES * NUM_SUBCORES

CHUNK = 128
CHUNKS_PER_TILE = -(-N_EDGES // (NUM_TILES * CHUNK))
EDGES_PER_TILE = CHUNKS_PER_TILE * CHUNK
E_PAD = EDGES_PER_TILE * NUM_TILES

ACC_ROWS = 10016
ROWS_PER_SUBCORE = ACC_ROWS // NUM_SUBCORES
DUMMY_ROW = N_NODES

DEG_W = 16


def _sc_body(emb_hbm, src_hbm, dst_hbm, ones_hbm, zeros_hbm, z16_hbm,
             part_hbm, deg_hbm,
             src_v, dst_v, rows_v, ones_v, zb_v, z16_v, acc_sh, deg_sh, sem):
  c = lax.axis_index("c")
  s = lax.axis_index("s")
  w = c * NUM_SUBCORES + s

  pltpu.sync_copy(ones_hbm, ones_v)
  pltpu.sync_copy(zeros_hbm, zb_v)
  pltpu.sync_copy(z16_hbm, z16_v)

  base = s * ROWS_PER_SUBCORE
  for k in range(ROWS_PER_SUBCORE // CHUNK):
    pltpu.sync_copy(zb_v, acc_sh.at[pl.ds(base + k * CHUNK, CHUNK)])
    pltpu.sync_copy(z16_v, deg_sh.at[pl.ds(base + k * CHUNK, CHUNK)])
  rem = ROWS_PER_SUBCORE % CHUNK
  if rem:
    off = base + (ROWS_PER_SUBCORE // CHUNK) * CHUNK
    pltpu.sync_copy(zb_v.at[pl.ds(0, rem)], acc_sh.at[pl.ds(off, rem)])
    pltpu.sync_copy(z16_v.at[pl.ds(0, rem)], deg_sh.at[pl.ds(off, rem)])

  plsc.subcore_barrier()

  pltpu.sync_copy(src_hbm.at[w], src_v)
  pltpu.sync_copy(dst_hbm.at[w], dst_v)

  @pl.loop(0, CHUNKS_PER_TILE)
  def _(j):
    pltpu.async_copy(emb_hbm.at[src_v.at[j]], rows_v, sem).wait()
    pltpu.sync_copy(rows_v, acc_sh.at[dst_v.at[j]], add=True)
    pltpu.sync_copy(ones_v, deg_sh.at[dst_v.at[j]], add=True)

  plsc.subcore_barrier()

  pltpu.sync_copy(acc_sh.at[pl.ds(base, ROWS_PER_SUBCORE)],
                  part_hbm.at[c, pl.ds(base, ROWS_PER_SUBCORE)])
  pltpu.sync_copy(deg_sh.at[pl.ds(base, ROWS_PER_SUBCORE)],
                  deg_hbm.at[c, pl.ds(base, ROWS_PER_SUBCORE)])


def _combine_body(p0_ref, p1_ref, d0_ref, d1_ref, o_ref):
  deg = d0_ref[:, 0:1] + d1_ref[:, 0:1]
  o_ref[...] = (p0_ref[...] + p1_ref[...]) / jnp.maximum(deg, 1.0)


@jax.jit
def kernel(user_emb, edge_index):
  src = edge_index[0].astype(jnp.int32)
  dst = edge_index[1].astype(jnp.int32)
  pad = E_PAD - N_EDGES
  src = jnp.concatenate([src, jnp.zeros((pad,), jnp.int32)])
  dst = jnp.concatenate([dst, jnp.full((pad,), DUMMY_ROW, jnp.int32)])
  src_r = src.reshape(NUM_TILES, CHUNKS_PER_TILE, CHUNK)
  dst_r = dst.reshape(NUM_TILES, CHUNKS_PER_TILE, CHUNK)

  ones16 = jnp.ones((CHUNK, DEG_W), jnp.float32)
  z128 = jnp.zeros((CHUNK, D_FEAT), jnp.float32)
  z16 = jnp.zeros((CHUNK, DEG_W), jnp.float32)

  mesh = plsc.VectorSubcoreMesh(core_axis_name="c", subcore_axis_name="s")
  sc = pl.kernel(
      _sc_body,
      out_type=[
          jax.ShapeDtypeStruct((NUM_CORES, ACC_ROWS, D_FEAT), jnp.float32),
          jax.ShapeDtypeStruct((NUM_CORES, ACC_ROWS, DEG_W), jnp.float32),
      ],
      mesh=mesh,
      scratch_types=[
          pltpu.VMEM((CHUNKS_PER_TILE, CHUNK), jnp.int32),
          pltpu.VMEM((CHUNKS_PER_TILE, CHUNK), jnp.int32),
          pltpu.VMEM((CHUNK, D_FEAT), jnp.float32),
          pltpu.VMEM((CHUNK, DEG_W), jnp.float32),
          pltpu.VMEM((CHUNK, D_FEAT), jnp.float32),
          pltpu.VMEM((CHUNK, DEG_W), jnp.float32),
          pltpu.VMEM_SHARED((ACC_ROWS, D_FEAT), jnp.float32),
          pltpu.VMEM_SHARED((ACC_ROWS, DEG_W), jnp.float32),
          pltpu.SemaphoreType.DMA,
      ],
  )
  part, deg = sc(user_emb, src_r, dst_r, ones16, z128, z16)

  out = pl.pallas_call(
      _combine_body,
      out_shape=jax.ShapeDtypeStruct((N_NODES, D_FEAT), jnp.float32),
  )(part[0, :N_NODES], part[1, :N_NODES], deg[0, :N_NODES], deg[1, :N_NODES])
  return out


# speedup vs baseline: 6.4091x; 6.4091x over previous
import jax
import jax.numpy as jnp
from jax import lax
from jax.experimental import pallas as pl
from jax.experimental.pallas import tpu as pltpu
from jax.experimental.pallas import tpu_sc as plsc

N_NODES = 10000
N_EDGES = 320000
D_FEAT = 128

NUM_CORES = 2
NUM_SUBCORES = 16
D_HALF = D_FEAT // NUM_CORES

CHUNK = 128
CHUNKS_PER_TILE = -(-N_EDGES // (NUM_SUBCORES * CHUNK))
EDGES_PER_TILE = CHUNKS_PER_TILE * CHUNK
E_PAD = EDGES_PER_TILE * NUM_SUBCORES

ACC_ROWS = 10112
ROWS_PER_SUBCORE = ACC_ROWS // NUM_SUBCORES
DUMMY_ROW = N_NODES

DEG_W = 16


def _sc_body(emb_hbm, src_hbm, dst_hbm, ones_hbm, zeros_hbm, z16_hbm,
             part_hbm, deg_hbm,
             src_v, dst_v, rows_v, ones_v, zb_v, z16_v, acc_sh, deg_sh, sem):
  c = lax.axis_index("c")
  s = lax.axis_index("s")

  pltpu.sync_copy(ones_hbm, ones_v)
  pltpu.sync_copy(zeros_hbm, zb_v)
  pltpu.sync_copy(z16_hbm, z16_v)

  base = s * ROWS_PER_SUBCORE
  for k in range(ROWS_PER_SUBCORE // CHUNK):
    pltpu.sync_copy(zb_v, acc_sh.at[pl.ds(base + k * CHUNK, CHUNK)])
    pltpu.sync_copy(z16_v, deg_sh.at[pl.ds(base + k * CHUNK, CHUNK)])
  rem = ROWS_PER_SUBCORE % CHUNK
  if rem:
    off = base + (ROWS_PER_SUBCORE // CHUNK) * CHUNK
    pltpu.sync_copy(zb_v.at[pl.ds(0, rem)], acc_sh.at[pl.ds(off, rem)])
    pltpu.sync_copy(z16_v.at[pl.ds(0, rem)], deg_sh.at[pl.ds(off, rem)])

  plsc.subcore_barrier()

  pltpu.sync_copy(src_hbm.at[s], src_v)
  pltpu.sync_copy(dst_hbm.at[s], dst_v)

  @pl.loop(0, CHUNKS_PER_TILE)
  def _(j):
    pltpu.async_copy(emb_hbm.at[c].at[src_v.at[j]], rows_v, sem).wait()
    pltpu.sync_copy(rows_v, acc_sh.at[dst_v.at[j]], add=True)

    @pl.when(c == 0)
    def _():
      pltpu.sync_copy(ones_v, deg_sh.at[dst_v.at[j]], add=True)

  plsc.subcore_barrier()

  pltpu.sync_copy(acc_sh.at[pl.ds(base, ROWS_PER_SUBCORE)],
                  part_hbm.at[c, pl.ds(base, ROWS_PER_SUBCORE)])

  @pl.when(c == 0)
  def _():
    pltpu.sync_copy(deg_sh.at[pl.ds(base, ROWS_PER_SUBCORE)],
                    deg_hbm.at[pl.ds(base, ROWS_PER_SUBCORE)])


def _combine_body(p0_ref, p1_ref, d_ref, o_ref):
  deg = jnp.maximum(d_ref[:, 0:1], 1.0)
  o_ref[...] = jnp.concatenate([p0_ref[...], p1_ref[...]], axis=1) / deg


@jax.jit
def kernel(user_emb, edge_index):
  src = edge_index[0].astype(jnp.int32)
  dst = edge_index[1].astype(jnp.int32)
  pad = E_PAD - N_EDGES
  src = jnp.concatenate([src, jnp.zeros((pad,), jnp.int32)])
  dst = jnp.concatenate([dst, jnp.full((pad,), DUMMY_ROW, jnp.int32)])
  src_r = src.reshape(NUM_SUBCORES, CHUNKS_PER_TILE, CHUNK)
  dst_r = dst.reshape(NUM_SUBCORES, CHUNKS_PER_TILE, CHUNK)

  emb_t = user_emb.reshape(N_NODES, NUM_CORES, D_HALF).transpose(1, 0, 2)

  ones16 = jnp.ones((CHUNK, DEG_W), jnp.float32)
  zhalf = jnp.zeros((CHUNK, D_HALF), jnp.float32)
  z16 = jnp.zeros((CHUNK, DEG_W), jnp.float32)

  mesh = plsc.VectorSubcoreMesh(core_axis_name="c", subcore_axis_name="s")
  sc = pl.kernel(
      _sc_body,
      out_type=[
          jax.ShapeDtypeStruct((NUM_CORES, ACC_ROWS, D_HALF), jnp.float32),
          jax.ShapeDtypeStruct((ACC_ROWS, DEG_W), jnp.float32),
      ],
      mesh=mesh,
      compiler_params=pltpu.CompilerParams(use_tc_tiling_on_sc=False),
      scratch_types=[
          pltpu.VMEM((CHUNKS_PER_TILE, CHUNK), jnp.int32),
          pltpu.VMEM((CHUNKS_PER_TILE, CHUNK), jnp.int32),
          pltpu.VMEM((CHUNK, D_HALF), jnp.float32),
          pltpu.VMEM((CHUNK, DEG_W), jnp.float32),
          pltpu.VMEM((CHUNK, D_HALF), jnp.float32),
          pltpu.VMEM((CHUNK, DEG_W), jnp.float32),
          pltpu.VMEM_SHARED((ACC_ROWS, D_HALF), jnp.float32),
          pltpu.VMEM_SHARED((ACC_ROWS, DEG_W), jnp.float32),
          pltpu.SemaphoreType.DMA,
      ],
  )
  part, deg = sc(emb_t, src_r, dst_r, ones16, zhalf, z16)

  out = pl.pallas_call(
      _combine_body,
      out_shape=jax.ShapeDtypeStruct((N_NODES, D_FEAT), jnp.float32),
  )(part[0, :N_NODES], part[1, :N_NODES], deg[:N_NODES])
  return out


# speedup vs baseline: 7.4905x; 1.1687x over previous
import jax
import jax.numpy as jnp
from jax import lax
from jax.experimental import pallas as pl
from jax.experimental.pallas import tpu as pltpu
from jax.experimental.pallas import tpu_sc as plsc

N_NODES = 10000
N_EDGES = 320000
D_FEAT = 128

NUM_CORES = 2
NUM_SUBCORES = 16
D_HALF = D_FEAT // NUM_CORES

CHUNK = 128
CHUNKS_PER_TILE = 2 * -(-N_EDGES // (NUM_SUBCORES * CHUNK * 2))
EDGES_PER_TILE = CHUNKS_PER_TILE * CHUNK
E_PAD = EDGES_PER_TILE * NUM_SUBCORES

ACC_ROWS = 10112
ROWS_PER_SUBCORE = ACC_ROWS // NUM_SUBCORES
DUMMY_ROW = N_NODES

DEG_W = 16


def _sc_body(emb_hbm, src_hbm, dst_hbm, ones_hbm, zeros_hbm, z16_hbm,
             part_hbm, deg_hbm,
             src_v, dst_v, rows0_v, rows1_v, ones_v, zb_v, z16_v,
             acc_sh, deg_sh, sem0, sem1):
  c = lax.axis_index("c")
  s = lax.axis_index("s")

  pltpu.sync_copy(ones_hbm, ones_v)
  pltpu.sync_copy(zeros_hbm, zb_v)
  pltpu.sync_copy(z16_hbm, z16_v)

  base = s * ROWS_PER_SUBCORE
  for k in range(ROWS_PER_SUBCORE // CHUNK):
    pltpu.sync_copy(zb_v, acc_sh.at[pl.ds(base + k * CHUNK, CHUNK)])
    pltpu.sync_copy(z16_v, deg_sh.at[pl.ds(base + k * CHUNK, CHUNK)])
  rem = ROWS_PER_SUBCORE % CHUNK
  if rem:
    off = base + (ROWS_PER_SUBCORE // CHUNK) * CHUNK
    pltpu.sync_copy(zb_v.at[pl.ds(0, rem)], acc_sh.at[pl.ds(off, rem)])
    pltpu.sync_copy(z16_v.at[pl.ds(0, rem)], deg_sh.at[pl.ds(off, rem)])

  plsc.subcore_barrier()

  pltpu.sync_copy(src_hbm.at[s], src_v)
  pltpu.sync_copy(dst_hbm.at[s], dst_v)

  def gather(j, buf, sem):
    return pltpu.async_copy(emb_hbm.at[c].at[src_v.at[j]], buf, sem)

  def consume(j, buf):
    pltpu.sync_copy(buf, acc_sh.at[dst_v.at[j]], add=True)

    @pl.when(c == 0)
    def _():
      pltpu.sync_copy(ones_v, deg_sh.at[dst_v.at[j]], add=True)

  gather(0, rows0_v, sem0)

  @pl.loop(0, CHUNKS_PER_TILE // 2)
  def _(i):
    j0 = 2 * i
    gather(j0 + 1, rows1_v, sem1)
    pltpu.make_async_copy(emb_hbm.at[c].at[src_v.at[j0]], rows0_v,
                          sem0).wait()
    consume(j0, rows0_v)

    @pl.when(j0 + 2 < CHUNKS_PER_TILE)
    def _():
      gather(j0 + 2, rows0_v, sem0)

    pltpu.make_async_copy(emb_hbm.at[c].at[src_v.at[j0 + 1]], rows1_v,
                          sem1).wait()
    consume(j0 + 1, rows1_v)

  plsc.subcore_barrier()

  pltpu.sync_copy(acc_sh.at[pl.ds(base, ROWS_PER_SUBCORE)],
                  part_hbm.at[c, pl.ds(base, ROWS_PER_SUBCORE)])

  @pl.when(c == 0)
  def _():
    pltpu.sync_copy(deg_sh.at[pl.ds(base, ROWS_PER_SUBCORE)],
                    deg_hbm.at[pl.ds(base, ROWS_PER_SUBCORE)])


def _combine_body(p0_ref, p1_ref, d_ref, o_ref):
  deg = jnp.maximum(d_ref[:, 0:1], 1.0)
  o_ref[...] = jnp.concatenate([p0_ref[...], p1_ref[...]], axis=1) / deg


@jax.jit
def kernel(user_emb, edge_index):
  src = edge_index[0].astype(jnp.int32)
  dst = edge_index[1].astype(jnp.int32)
  pad = E_PAD - N_EDGES
  src = jnp.concatenate([src, jnp.zeros((pad,), jnp.int32)])
  dst = jnp.concatenate([dst, jnp.full((pad,), DUMMY_ROW, jnp.int32)])
  src_r = src.reshape(NUM_SUBCORES, CHUNKS_PER_TILE, CHUNK)
  dst_r = dst.reshape(NUM_SUBCORES, CHUNKS_PER_TILE, CHUNK)

  emb_t = user_emb.reshape(N_NODES, NUM_CORES, D_HALF).transpose(1, 0, 2)

  ones16 = jnp.ones((CHUNK, DEG_W), jnp.float32)
  zhalf = jnp.zeros((CHUNK, D_HALF), jnp.float32)
  z16 = jnp.zeros((CHUNK, DEG_W), jnp.float32)

  mesh = plsc.VectorSubcoreMesh(core_axis_name="c", subcore_axis_name="s")
  sc = pl.kernel(
      _sc_body,
      out_type=[
          jax.ShapeDtypeStruct((NUM_CORES, ACC_ROWS, D_HALF), jnp.float32),
          jax.ShapeDtypeStruct((ACC_ROWS, DEG_W), jnp.float32),
      ],
      mesh=mesh,
      compiler_params=pltpu.CompilerParams(use_tc_tiling_on_sc=False),
      scratch_types=[
          pltpu.VMEM((CHUNKS_PER_TILE, CHUNK), jnp.int32),
          pltpu.VMEM((CHUNKS_PER_TILE, CHUNK), jnp.int32),
          pltpu.VMEM((CHUNK, D_HALF), jnp.float32),
          pltpu.VMEM((CHUNK, D_HALF), jnp.float32),
          pltpu.VMEM((CHUNK, DEG_W), jnp.float32),
          pltpu.VMEM((CHUNK, D_HALF), jnp.float32),
          pltpu.VMEM((CHUNK, DEG_W), jnp.float32),
          pltpu.VMEM_SHARED((ACC_ROWS, D_HALF), jnp.float32),
          pltpu.VMEM_SHARED((ACC_ROWS, DEG_W), jnp.float32),
          pltpu.SemaphoreType.DMA,
          pltpu.SemaphoreType.DMA,
      ],
  )
  part, deg = sc(emb_t, src_r, dst_r, ones16, zhalf, z16)

  out = pl.pallas_call(
      _combine_body,
      out_shape=jax.ShapeDtypeStruct((N_NODES, D_FEAT), jnp.float32),
  )(part[0, :N_NODES], part[1, :N_NODES], deg[:N_NODES])
  return out
